# Initial kernel scaffold; baseline (speedup 1.0000x reference)
import jax
import jax.numpy as jnp
from jax.experimental import pallas as pl


def kernel(state_x, state_edge_index, state_edge_type, state_batch, goal_x, goal_edge_index, goal_edge_type, goal_batch, depth, s1_W, s1_root, s1_b, s2_W, s2_root, s2_b, g1_W, g1_root, g1_b, g2_W, g2_root, g2_b, reg_W1, reg_b1, reg_W2, reg_b2):
    raise NotImplementedError("write your pallas kernel here")


# speedup vs baseline: 1.4416x; 1.4416x over previous
import functools

import jax
import jax.numpy as jnp
from jax import lax
from jax.experimental import pallas as pl
from jax.experimental.pallas import tpu as pltpu
from jax.experimental.pallas import tpu_sc as plsc

NN = 10000
EE = 320000
DD = 128
HH = 64
RR = 32
BB = 64

NC = 2
NS = 16
LL = 16
NW = NC * NS

KK = 80
EW = EE // NW
EC = EE // NS
NRR = NN * RR
ZB = 2000
ZR = 40
NCH = NN // ZR
CPT = -(-NCH // NS)

_mesh = plsc.VectorSubcoreMesh(core_axis_name="c", subcore_axis_name="s",
                               num_cores=NC, num_subcores=NS)
_sc_params = pltpu.CompilerParams(use_tc_tiling_on_sc=False)


def _sc_norm_body(src_hbm, dst_hbm, et_hbm, combsrc_hbm, norm_hbm,
                  counts, src_v, dst_v, et_v, comb_v, comb2_v, ones_v,
                  cnt_v, norm_v, zbuf, sem):
    c = lax.axis_index("c")
    s = lax.axis_index("s")
    w = s * NC + c

    for j in range(KK // LL):
        ones_v[pl.ds(j * LL, LL)] = jnp.ones((LL,), jnp.float32)

    def zfill(i, _):
        zbuf[pl.ds(i * LL, LL)] = jnp.zeros((LL,), jnp.float32)
        return 0
    lax.fori_loop(0, ZB // LL, zfill, 0)

    def zcopy(j, _):
        pltpu.sync_copy(zbuf, counts.at[pl.ds(s * (NRR // NS) + j * ZB, ZB)])
        return 0
    lax.fori_loop(0, (NRR // NS) // ZB, zcopy, 0)
    plsc.subcore_barrier()

    def p1(i, _):
        base = s * EC + i * KK
        pltpu.sync_copy(dst_hbm.at[pl.ds(base, KK)], dst_v)
        pltpu.sync_copy(et_hbm.at[pl.ds(base, KK)], et_v)
        for j in range(KK // LL):
            sl = pl.ds(j * LL, LL)
            comb_v[sl] = dst_v[sl] * RR + et_v[sl]
        pltpu.sync_copy(ones_v, counts.at[comb_v], add=True)

        @pl.when(c == 0)
        def _():
            pltpu.sync_copy(src_hbm.at[pl.ds(base, KK)], src_v)
            for j in range(KK // LL):
                sl = pl.ds(j * LL, LL)
                comb2_v[sl] = src_v[sl] * RR + et_v[sl]
            pltpu.sync_copy(comb2_v, combsrc_hbm.at[pl.ds(base, KK)])
        return 0
    lax.fori_loop(0, EC // KK, p1, 0)
    plsc.subcore_barrier()

    def p2(i, _):
        base = w * EW + i * KK
        pltpu.sync_copy(dst_hbm.at[pl.ds(base, KK)], dst_v)
        pltpu.sync_copy(et_hbm.at[pl.ds(base, KK)], et_v)
        for j in range(KK // LL):
            sl = pl.ds(j * LL, LL)
            comb_v[sl] = dst_v[sl] * RR + et_v[sl]
        pltpu.async_copy(counts.at[comb_v], cnt_v, sem).wait()
        for j in range(KK // LL):
            sl = pl.ds(j * LL, LL)
            norm_v[sl] = 1.0 / jnp.maximum(cnt_v[sl], 1.0)
        pltpu.sync_copy(norm_v, norm_hbm.at[pl.ds(base, KK)])
        return 0
    lax.fori_loop(0, EW // KK, p2, 0)


@functools.partial(
    pl.kernel,
    out_type=[jax.ShapeDtypeStruct((EE,), jnp.int32),
              jax.ShapeDtypeStruct((EE,), jnp.float32)],
    mesh=_mesh,
    compiler_params=_sc_params,
    scratch_types=[
        pltpu.VMEM_SHARED((NRR,), jnp.float32),
        pltpu.VMEM((KK,), jnp.int32),
        pltpu.VMEM((KK,), jnp.int32),
        pltpu.VMEM((KK,), jnp.int32),
        pltpu.VMEM((KK,), jnp.int32),
        pltpu.VMEM((KK,), jnp.int32),
        pltpu.VMEM((KK,), jnp.float32),
        pltpu.VMEM((KK,), jnp.float32),
        pltpu.VMEM((KK,), jnp.float32),
        pltpu.VMEM((ZB,), jnp.float32),
        pltpu.SemaphoreType.DMA,
    ],
)
def _sc_norm(src_hbm, dst_hbm, et_hbm, combsrc_hbm, norm_hbm, *rest):
    _sc_norm_body(src_hbm, dst_hbm, et_hbm, combsrc_hbm, norm_hbm, *rest)


def _sc_agg_body(xrel_hbm, dst_hbm, combsrc_hbm, norm_hbm, aggp_hbm,
                 agg, idx_v, dst_v, norm_v, rows_v, zrow, sem):
    c = lax.axis_index("c")
    s = lax.axis_index("s")
    w = s * NC + c

    for r in range(ZR):
        for j in range(HH // LL):
            zrow[r, pl.ds(j * LL, LL)] = jnp.zeros((LL,), jnp.float32)

    def zc(jj, _):
        j = jj * NS + s

        @pl.when(j < NCH)
        def _():
            pltpu.sync_copy(zrow, agg.at[pl.ds(j * ZR, ZR)])
        return 0
    lax.fori_loop(0, CPT, zc, 0)
    plsc.subcore_barrier()

    def body(i, _):
        base = w * EW + i * KK
        pltpu.sync_copy(combsrc_hbm.at[pl.ds(base, KK)], idx_v)
        pltpu.sync_copy(dst_hbm.at[pl.ds(base, KK)], dst_v)
        pltpu.sync_copy(norm_hbm.at[pl.ds(base, KK)], norm_v)
        pltpu.async_copy(xrel_hbm.at[idx_v], rows_v, sem).wait()
        for j16 in range(KK // LL):
            nv16 = norm_v[pl.ds(j16 * LL, LL)]
            for l in range(LL):
                e = j16 * LL + l
                nv = nv16[l]
                for j in range(HH // LL):
                    sl = pl.ds(j * LL, LL)
                    rows_v[e, sl] = rows_v[e, sl] * nv
        pltpu.sync_copy(rows_v, agg.at[dst_v], add=True)
        return 0
    lax.fori_loop(0, EW // KK, body, 0)
    plsc.subcore_barrier()

    def oc(jj, _):
        j = jj * NS + s

        @pl.when(j < NCH)
        def _():
            pltpu.sync_copy(agg.at[pl.ds(j * ZR, ZR)],
                            aggp_hbm.at[pl.ds(c * NN + j * ZR, ZR)])
        return 0
    lax.fori_loop(0, CPT, oc, 0)


@functools.partial(
    pl.kernel,
    out_type=jax.ShapeDtypeStruct((NC * NN, HH), jnp.float32),
    mesh=_mesh,
    compiler_params=_sc_params,
    scratch_types=[
        pltpu.VMEM_SHARED((NN, HH), jnp.float32),
        pltpu.VMEM((KK,), jnp.int32),
        pltpu.VMEM((KK,), jnp.int32),
        pltpu.VMEM((KK,), jnp.float32),
        pltpu.VMEM((KK, HH), jnp.float32),
        pltpu.VMEM((ZR, HH), jnp.float32),
        pltpu.SemaphoreType.DMA,
    ],
)
def _sc_agg(xrel_hbm, dst_hbm, combsrc_hbm, norm_hbm, aggp_hbm, *rest):
    _sc_agg_body(xrel_hbm, dst_hbm, combsrc_hbm, norm_hbm, aggp_hbm, *rest)


MM = 400
GG = NN // MM


def _mm1_body(x_ref, wr_ref, rt_ref, xrel_ref, rto_ref):
    x = x_ref[...]
    xrel_ref[...] = jnp.dot(x, wr_ref[...], preferred_element_type=jnp.float32)
    rto_ref[...] = jnp.dot(x, rt_ref[...], preferred_element_type=jnp.float32)


def _tc_mm1(x, wr, root):
    din, dout = wr.shape
    return pl.pallas_call(
        _mm1_body,
        grid=(GG,),
        in_specs=[pl.BlockSpec((MM, din), lambda i: (i, 0)),
                  pl.BlockSpec((din, dout), lambda i: (0, 0)),
                  pl.BlockSpec((din, HH), lambda i: (0, 0))],
        out_specs=[pl.BlockSpec((MM, dout), lambda i: (i, 0)),
                   pl.BlockSpec((MM, HH), lambda i: (i, 0))],
        out_shape=[jax.ShapeDtypeStruct((NN, dout), jnp.float32),
                   jax.ShapeDtypeStruct((NN, HH), jnp.float32)],
    )(x, wr, root)


def _mm2_body(aggp_ref, rt1_ref, b1_ref, wr_ref, rt2_ref, xrel_ref, rto_ref):
    h = jnp.maximum(aggp_ref[0] + aggp_ref[1] + rt1_ref[...] + b1_ref[...],
                    0.0)
    xrel_ref[...] = jnp.dot(h, wr_ref[...], preferred_element_type=jnp.float32)
    rto_ref[...] = jnp.dot(h, rt2_ref[...], preferred_element_type=jnp.float32)


def _tc_mm2(aggp, rt1, b1, wr, root):
    din, dout = wr.shape
    return pl.pallas_call(
        _mm2_body,
        grid=(GG,),
        in_specs=[pl.BlockSpec((NC, MM, HH), lambda i: (0, i, 0)),
                  pl.BlockSpec((MM, HH), lambda i: (i, 0)),
                  pl.BlockSpec((1, HH), lambda i: (0, 0)),
                  pl.BlockSpec((din, dout), lambda i: (0, 0)),
                  pl.BlockSpec((din, HH), lambda i: (0, 0))],
        out_specs=[pl.BlockSpec((MM, dout), lambda i: (i, 0)),
                   pl.BlockSpec((MM, HH), lambda i: (i, 0))],
        out_shape=[jax.ShapeDtypeStruct((NN, dout), jnp.float32),
                   jax.ShapeDtypeStruct((NN, HH), jnp.float32)],
    )(aggp, rt1, b1, wr, root)


def _pool_body(aggp_ref, rt2_ref, b2_ref, batch_ref, psum_ref, pcnt_ref):
    i = pl.program_id(0)
    h = jnp.maximum(aggp_ref[0] + aggp_ref[1] + rt2_ref[...] + b2_ref[...],
                    0.0)
    bt = batch_ref[0, 0, :]
    oh = (bt[None, :] == lax.broadcasted_iota(jnp.int32, (BB, MM), 0)
          ).astype(jnp.float32)
    ps = jnp.dot(oh, h, preferred_element_type=jnp.float32)
    pc = jnp.sum(oh, axis=1)[None, :]

    @pl.when(i == 0)
    def _():
        psum_ref[...] = jnp.zeros_like(psum_ref)
        pcnt_ref[...] = jnp.zeros_like(pcnt_ref)
    psum_ref[...] += ps
    pcnt_ref[...] += pc


def _tc_pool(aggp, rt2, b2, batch3d):
    return pl.pallas_call(
        _pool_body,
        grid=(GG,),
        in_specs=[pl.BlockSpec((NC, MM, HH), lambda i: (0, i, 0)),
                  pl.BlockSpec((MM, HH), lambda i: (i, 0)),
                  pl.BlockSpec((1, HH), lambda i: (0, 0)),
                  pl.BlockSpec((1, 1, MM), lambda i: (i, 0, 0))],
        out_specs=[pl.BlockSpec((BB, HH), lambda i: (0, 0)),
                   pl.BlockSpec((1, BB), lambda i: (0, 0))],
        out_shape=[jax.ShapeDtypeStruct((BB, HH), jnp.float32),
                   jax.ShapeDtypeStruct((1, BB), jnp.float32)],
    )(aggp, rt2, b2, batch3d)


def _tail_body(sps_ref, spc_ref, gps_ref, gpc_ref, d_ref, w1a_ref, w1b_ref,
               w1c_ref, b1_ref, w2r_ref, b2_ref, out_ref):
    se = sps_ref[...] / jnp.maximum(spc_ref[...], 1.0)
    ge = gps_ref[...] / jnp.maximum(gpc_ref[...], 1.0)
    d = d_ref[...]
    dm = jnp.mean(d)
    sd = jnp.sqrt(jnp.mean((d - dm) ** 2))
    dn = (d - dm) / (sd + 1e-6)
    z = (jnp.dot(se, w1a_ref[...], preferred_element_type=jnp.float32)
         + jnp.dot(ge, w1b_ref[...], preferred_element_type=jnp.float32)
         + dn * w1c_ref[...] + b1_ref[...])
    hh = jnp.maximum(z, 0.0)
    out_ref[...] = jnp.sum(hh * w2r_ref[...], axis=1, keepdims=True) \
        + b2_ref[...]


def _tc_tail(sps, spc, gps, gpc, d, w1a, w1b, w1c, b1, w2r, b2):
    return pl.pallas_call(
        _tail_body,
        out_shape=jax.ShapeDtypeStruct((BB, 1), jnp.float32),
    )(sps, spc, gps, gpc, d, w1a, w1b, w1c, b1, w2r, b2)


def _encode(x, src, dst, et, batch3d, W1, root1, b1, W2, root2, b2):
    comb_src, norm = _sc_norm(src, dst, et)
    wr1 = W1.transpose(1, 0, 2).reshape(W1.shape[1], RR * HH)
    wr2 = W2.transpose(1, 0, 2).reshape(W2.shape[1], RR * HH)
    xrel1, rt1 = _tc_mm1(x, wr1, root1)
    aggp1 = _sc_agg(xrel1.reshape(NRR, HH), dst, comb_src, norm)
    xrel2, rt2 = _tc_mm2(aggp1.reshape(NC, NN, HH), rt1,
                         b1.reshape(1, HH), wr2, root2)
    aggp2 = _sc_agg(xrel2.reshape(NRR, HH), dst, comb_src, norm)
    return _tc_pool(aggp2.reshape(NC, NN, HH), rt2, b2.reshape(1, HH),
                    batch3d)


def kernel(state_x, state_edge_index, state_edge_type, state_batch,
           goal_x, goal_edge_index, goal_edge_type, goal_batch, depth,
           s1_W, s1_root, s1_b, s2_W, s2_root, s2_b,
           g1_W, g1_root, g1_b, g2_W, g2_root, g2_b,
           reg_W1, reg_b1, reg_W2, reg_b2):
    s_sum, s_cnt = _encode(state_x, state_edge_index[0], state_edge_index[1],
                           state_edge_type, state_batch.reshape(GG, 1, MM),
                           s1_W, s1_root, s1_b, s2_W, s2_root, s2_b)
    g_sum, g_cnt = _encode(goal_x, goal_edge_index[0], goal_edge_index[1],
                           goal_edge_type, goal_batch.reshape(GG, 1, MM),
                           g1_W, g1_root, g1_b, g2_W, g2_root, g2_b)
    pred = _tc_tail(s_sum, s_cnt.reshape(BB, 1), g_sum, g_cnt.reshape(BB, 1),
                    depth.reshape(BB, 1),
                    reg_W1[:HH], reg_W1[HH:2 * HH], reg_W1[2 * HH:],
                    reg_b1.reshape(1, HH), reg_W2.reshape(1, HH),
                    reg_b2.reshape(1, 1))
    return pred.reshape(BB)


# speedup vs baseline: 3.5408x; 2.4562x over previous
import functools

import jax
import jax.numpy as jnp
from jax import lax
from jax.experimental import pallas as pl
from jax.experimental.pallas import tpu as pltpu
from jax.experimental.pallas import tpu_sc as plsc

NN = 10000
EE = 320000
DD = 128
HH = 64
RR = 32
BB = 64

NC = 2
NS = 16
LL = 16
NW = NC * NS

KK = 80
NB = 5
EW = EE // NW
EC = EE // NS
NRR = NN * RR
ZB = 2000
ZR = 40
NCH = NN // ZR
CPT = -(-NCH // NS)

_mesh = plsc.VectorSubcoreMesh(core_axis_name="c", subcore_axis_name="s",
                               num_cores=NC, num_subcores=NS)
_sc_params = pltpu.CompilerParams(use_tc_tiling_on_sc=False)


def _sc_norm_body(src_hbm, dst_hbm, et_hbm, combsrc_hbm, norm_hbm,
                  counts, src_vs, dst_vs, et_vs, comb_vs, comb2_vs, ones_v,
                  cnt_vs, norm_vs, zbuf, *sems):
    lsems = sems[0:NB]
    gsems = sems[NB:2 * NB]
    ssems = sems[2 * NB:3 * NB]
    csems = sems[3 * NB:4 * NB]
    c = lax.axis_index("c")
    s = lax.axis_index("s")
    w = s * NC + c

    for j in range(KK // LL):
        ones_v[pl.ds(j * LL, LL)] = jnp.ones((LL,), jnp.float32)

    def zfill(i, _):
        zbuf[pl.ds(i * LL, LL)] = jnp.zeros((LL,), jnp.float32)
        return 0
    lax.fori_loop(0, ZB // LL, zfill, 0)

    def zcopy(j, _):
        pltpu.sync_copy(zbuf, counts.at[pl.ds(s * (NRR // NS) + j * ZB, ZB)])
        return 0
    lax.fori_loop(0, (NRR // NS) // ZB, zcopy, 0)
    plsc.subcore_barrier()

    def p1(g, _):
        base0 = s * EC + g * (NB * KK)
        ldescs = []
        for b in range(NB):
            base = base0 + b * KK
            d1 = pltpu.async_copy(dst_hbm.at[pl.ds(base, KK)],
                                  dst_vs.at[b], lsems[b])
            d2 = pltpu.async_copy(et_hbm.at[pl.ds(base, KK)],
                                  et_vs.at[b], lsems[b])
            ldescs.append((d1, d2))

            @pl.when(c == 0)
            def _():
                pltpu.async_copy(src_hbm.at[pl.ds(base, KK)],
                                 src_vs.at[b], csems[b])
        adescs = []
        for b in range(NB):
            for d in ldescs[b]:
                d.wait()
            for j in range(KK // LL):
                sl = pl.ds(j * LL, LL)
                comb_vs[b, sl] = dst_vs[b, sl] * RR + et_vs[b, sl]
            adescs.append(pltpu.async_copy(ones_v, counts.at[comb_vs.at[b]],
                                           gsems[b], add=True))

            @pl.when(c == 0)
            def _():
                base = base0 + b * KK
                pltpu.make_async_copy(src_hbm.at[pl.ds(base, KK)],
                                      src_vs.at[b], csems[b]).wait()
                for j in range(KK // LL):
                    sl = pl.ds(j * LL, LL)
                    comb2_vs[b, sl] = src_vs[b, sl] * RR + et_vs[b, sl]
                pltpu.async_copy(comb2_vs.at[b],
                                 combsrc_hbm.at[pl.ds(base, KK)], ssems[b])
        for b in range(NB):
            adescs[b].wait()

            @pl.when(c == 0)
            def _():
                base = base0 + b * KK
                pltpu.make_async_copy(comb2_vs.at[b],
                                      combsrc_hbm.at[pl.ds(base, KK)],
                                      ssems[b]).wait()
        return 0
    lax.fori_loop(0, EC // (NB * KK), p1, 0)
    plsc.subcore_barrier()

    def p2(g, _):
        base0 = w * EW + g * (NB * KK)
        ldescs = []
        for b in range(NB):
            base = base0 + b * KK
            d1 = pltpu.async_copy(dst_hbm.at[pl.ds(base, KK)],
                                  dst_vs.at[b], lsems[b])
            d2 = pltpu.async_copy(et_hbm.at[pl.ds(base, KK)],
                                  et_vs.at[b], lsems[b])
            ldescs.append((d1, d2))
        gdescs = []
        for b in range(NB):
            for d in ldescs[b]:
                d.wait()
            for j in range(KK // LL):
                sl = pl.ds(j * LL, LL)
                comb_vs[b, sl] = dst_vs[b, sl] * RR + et_vs[b, sl]
            gdescs.append(pltpu.async_copy(counts.at[comb_vs.at[b]],
                                           cnt_vs.at[b], gsems[b]))
        sdescs = []
        for b in range(NB):
            base = base0 + b * KK
            gdescs[b].wait()
            for j in range(KK // LL):
                sl = pl.ds(j * LL, LL)
                norm_vs[b, sl] = 1.0 / jnp.maximum(cnt_vs[b, sl], 1.0)
            sdescs.append(pltpu.async_copy(norm_vs.at[b],
                                           norm_hbm.at[pl.ds(base, KK)],
                                           ssems[b]))
        for d in sdescs:
            d.wait()
        return 0
    lax.fori_loop(0, EW // (NB * KK), p2, 0)


@functools.partial(
    pl.kernel,
    out_type=[jax.ShapeDtypeStruct((EE,), jnp.int32),
              jax.ShapeDtypeStruct((EE,), jnp.float32)],
    mesh=_mesh,
    compiler_params=_sc_params,
    scratch_types=[
        pltpu.VMEM_SHARED((NRR,), jnp.float32),
        pltpu.VMEM((NB, KK), jnp.int32),
        pltpu.VMEM((NB, KK), jnp.int32),
        pltpu.VMEM((NB, KK), jnp.int32),
        pltpu.VMEM((NB, KK), jnp.int32),
        pltpu.VMEM((NB, KK), jnp.int32),
        pltpu.VMEM((KK,), jnp.float32),
        pltpu.VMEM((NB, KK), jnp.float32),
        pltpu.VMEM((NB, KK), jnp.float32),
        pltpu.VMEM((ZB,), jnp.float32),
    ] + [pltpu.SemaphoreType.DMA] * (4 * NB),
)
def _sc_norm(src_hbm, dst_hbm, et_hbm, combsrc_hbm, norm_hbm, *rest):
    _sc_norm_body(src_hbm, dst_hbm, et_hbm, combsrc_hbm, norm_hbm, *rest)


def _sc_agg_body(xrel_hbm, dst_hbm, combsrc_hbm, norm_hbm, aggp_hbm,
                 agg, idx_vs, dst_vs, norm_vs, rows_vs, zrow, *sems):
    lsems = sems[0:NB]
    gsems = sems[NB:2 * NB]
    ssems = sems[2 * NB:3 * NB]
    c = lax.axis_index("c")
    s = lax.axis_index("s")
    w = s * NC + c

    for r in range(ZR):
        for j in range(HH // LL):
            zrow[r, pl.ds(j * LL, LL)] = jnp.zeros((LL,), jnp.float32)

    def zc(jj, _):
        j = jj * NS + s

        @pl.when(j < NCH)
        def _():
            pltpu.sync_copy(zrow, agg.at[pl.ds(j * ZR, ZR)])
        return 0
    lax.fori_loop(0, CPT, zc, 0)
    plsc.subcore_barrier()

    def group(g, _):
        base0 = w * EW + g * (NB * KK)
        ldescs = []
        for b in range(NB):
            base = base0 + b * KK
            d1 = pltpu.async_copy(combsrc_hbm.at[pl.ds(base, KK)],
                                  idx_vs.at[b], lsems[b])
            d2 = pltpu.async_copy(dst_hbm.at[pl.ds(base, KK)],
                                  dst_vs.at[b], lsems[b])
            d3 = pltpu.async_copy(norm_hbm.at[pl.ds(base, KK)],
                                  norm_vs.at[b], lsems[b])
            ldescs.append((d1, d2, d3))
        gdescs = []
        for b in range(NB):
            for d in ldescs[b]:
                d.wait()
            gdescs.append(pltpu.async_copy(xrel_hbm.at[idx_vs.at[b]],
                                           rows_vs.at[b], gsems[b]))
        sdescs = []
        for b in range(NB):
            gdescs[b].wait()
            for j16 in range(KK // LL):
                nv16 = norm_vs[b, pl.ds(j16 * LL, LL)]
                for l in range(LL):
                    e = j16 * LL + l
                    nv = nv16[l]
                    for j in range(HH // LL):
                        sl = pl.ds(j * LL, LL)
                        rows_vs[b, e, sl] = rows_vs[b, e, sl] * nv
            sdescs.append(pltpu.async_copy(rows_vs.at[b],
                                           agg.at[dst_vs.at[b]],
                                           ssems[b], add=True))
        for d in sdescs:
            d.wait()
        return 0
    lax.fori_loop(0, EW // (NB * KK), group, 0)
    plsc.subcore_barrier()

    def oc(jj, _):
        j = jj * NS + s

        @pl.when(j < NCH)
        def _():
            pltpu.sync_copy(agg.at[pl.ds(j * ZR, ZR)],
                            aggp_hbm.at[pl.ds(c * NN + j * ZR, ZR)])
        return 0
    lax.fori_loop(0, CPT, oc, 0)


@functools.partial(
    pl.kernel,
    out_type=jax.ShapeDtypeStruct((NC * NN, HH), jnp.float32),
    mesh=_mesh,
    compiler_params=_sc_params,
    scratch_types=[
        pltpu.VMEM_SHARED((NN, HH), jnp.float32),
        pltpu.VMEM((NB, KK), jnp.int32),
        pltpu.VMEM((NB, KK), jnp.int32),
        pltpu.VMEM((NB, KK), jnp.float32),
        pltpu.VMEM((NB, KK, HH), jnp.float32),
        pltpu.VMEM((ZR, HH), jnp.float32),
    ] + [pltpu.SemaphoreType.DMA] * (3 * NB),
)
def _sc_agg(xrel_hbm, dst_hbm, combsrc_hbm, norm_hbm, aggp_hbm, *rest):
    _sc_agg_body(xrel_hbm, dst_hbm, combsrc_hbm, norm_hbm, aggp_hbm, *rest)


MM = 400
GG = NN // MM


def _mm1_body(x_ref, wr_ref, rt_ref, xrel_ref, rto_ref):
    x = x_ref[...]
    xrel_ref[...] = jnp.dot(x, wr_ref[...], preferred_element_type=jnp.float32)
    rto_ref[...] = jnp.dot(x, rt_ref[...], preferred_element_type=jnp.float32)


def _tc_mm1(x, wr, root):
    din, dout = wr.shape
    return pl.pallas_call(
        _mm1_body,
        grid=(GG,),
        in_specs=[pl.BlockSpec((MM, din), lambda i: (i, 0)),
                  pl.BlockSpec((din, dout), lambda i: (0, 0)),
                  pl.BlockSpec((din, HH), lambda i: (0, 0))],
        out_specs=[pl.BlockSpec((MM, dout), lambda i: (i, 0)),
                   pl.BlockSpec((MM, HH), lambda i: (i, 0))],
        out_shape=[jax.ShapeDtypeStruct((NN, dout), jnp.float32),
                   jax.ShapeDtypeStruct((NN, HH), jnp.float32)],
    )(x, wr, root)


def _mm2_body(aggp_ref, rt1_ref, b1_ref, wr_ref, rt2_ref, xrel_ref, rto_ref):
    h = jnp.maximum(aggp_ref[0] + aggp_ref[1] + rt1_ref[...] + b1_ref[...],
                    0.0)
    xrel_ref[...] = jnp.dot(h, wr_ref[...], preferred_element_type=jnp.float32)
    rto_ref[...] = jnp.dot(h, rt2_ref[...], preferred_element_type=jnp.float32)


def _tc_mm2(aggp, rt1, b1, wr, root):
    din, dout = wr.shape
    return pl.pallas_call(
        _mm2_body,
        grid=(GG,),
        in_specs=[pl.BlockSpec((NC, MM, HH), lambda i: (0, i, 0)),
                  pl.BlockSpec((MM, HH), lambda i: (i, 0)),
                  pl.BlockSpec((1, HH), lambda i: (0, 0)),
                  pl.BlockSpec((din, dout), lambda i: (0, 0)),
                  pl.BlockSpec((din, HH), lambda i: (0, 0))],
        out_specs=[pl.BlockSpec((MM, dout), lambda i: (i, 0)),
                   pl.BlockSpec((MM, HH), lambda i: (i, 0))],
        out_shape=[jax.ShapeDtypeStruct((NN, dout), jnp.float32),
                   jax.ShapeDtypeStruct((NN, HH), jnp.float32)],
    )(aggp, rt1, b1, wr, root)


def _pool_body(aggp_ref, rt2_ref, b2_ref, batch_ref, psum_ref, pcnt_ref):
    i = pl.program_id(0)
    h = jnp.maximum(aggp_ref[0] + aggp_ref[1] + rt2_ref[...] + b2_ref[...],
                    0.0)
    bt = batch_ref[0, 0, :]
    oh = (bt[None, :] == lax.broadcasted_iota(jnp.int32, (BB, MM), 0)
          ).astype(jnp.float32)
    ps = jnp.dot(oh, h, preferred_element_type=jnp.float32)
    pc = jnp.sum(oh, axis=1)[None, :]

    @pl.when(i == 0)
    def _():
        psum_ref[...] = jnp.zeros_like(psum_ref)
        pcnt_ref[...] = jnp.zeros_like(pcnt_ref)
    psum_ref[...] += ps
    pcnt_ref[...] += pc


def _tc_pool(aggp, rt2, b2, batch3d):
    return pl.pallas_call(
        _pool_body,
        grid=(GG,),
        in_specs=[pl.BlockSpec((NC, MM, HH), lambda i: (0, i, 0)),
                  pl.BlockSpec((MM, HH), lambda i: (i, 0)),
                  pl.BlockSpec((1, HH), lambda i: (0, 0)),
                  pl.BlockSpec((1, 1, MM), lambda i: (i, 0, 0))],
        out_specs=[pl.BlockSpec((BB, HH), lambda i: (0, 0)),
                   pl.BlockSpec((1, BB), lambda i: (0, 0))],
        out_shape=[jax.ShapeDtypeStruct((BB, HH), jnp.float32),
                   jax.ShapeDtypeStruct((1, BB), jnp.float32)],
    )(aggp, rt2, b2, batch3d)


def _tail_body(sps_ref, spc_ref, gps_ref, gpc_ref, d_ref, w1a_ref, w1b_ref,
               w1c_ref, b1_ref, w2r_ref, b2_ref, out_ref):
    se = sps_ref[...] / jnp.maximum(spc_ref[...], 1.0)
    ge = gps_ref[...] / jnp.maximum(gpc_ref[...], 1.0)
    d = d_ref[...]
    dm = jnp.mean(d)
    sd = jnp.sqrt(jnp.mean((d - dm) ** 2))
    dn = (d - dm) / (sd + 1e-6)
    z = (jnp.dot(se, w1a_ref[...], preferred_element_type=jnp.float32)
         + jnp.dot(ge, w1b_ref[...], preferred_element_type=jnp.float32)
         + dn * w1c_ref[...] + b1_ref[...])
    hh = jnp.maximum(z, 0.0)
    out_ref[...] = jnp.sum(hh * w2r_ref[...], axis=1, keepdims=True) \
        + b2_ref[...]


def _tc_tail(sps, spc, gps, gpc, d, w1a, w1b, w1c, b1, w2r, b2):
    return pl.pallas_call(
        _tail_body,
        out_shape=jax.ShapeDtypeStruct((BB, 1), jnp.float32),
    )(sps, spc, gps, gpc, d, w1a, w1b, w1c, b1, w2r, b2)


def _encode(x, src, dst, et, batch3d, W1, root1, b1, W2, root2, b2):
    comb_src, norm = _sc_norm(src, dst, et)
    wr1 = W1.transpose(1, 0, 2).reshape(W1.shape[1], RR * HH)
    wr2 = W2.transpose(1, 0, 2).reshape(W2.shape[1], RR * HH)
    xrel1, rt1 = _tc_mm1(x, wr1, root1)
    aggp1 = _sc_agg(xrel1.reshape(NRR, HH), dst, comb_src, norm)
    xrel2, rt2 = _tc_mm2(aggp1.reshape(NC, NN, HH), rt1,
                         b1.reshape(1, HH), wr2, root2)
    aggp2 = _sc_agg(xrel2.reshape(NRR, HH), dst, comb_src, norm)
    return _tc_pool(aggp2.reshape(NC, NN, HH), rt2, b2.reshape(1, HH),
                    batch3d)


def kernel(state_x, state_edge_index, state_edge_type, state_batch,
           goal_x, goal_edge_index, goal_edge_type, goal_batch, depth,
           s1_W, s1_root, s1_b, s2_W, s2_root, s2_b,
           g1_W, g1_root, g1_b, g2_W, g2_root, g2_b,
           reg_W1, reg_b1, reg_W2, reg_b2):
    s_sum, s_cnt = _encode(state_x, state_edge_index[0], state_edge_index[1],
                           state_edge_type, state_batch.reshape(GG, 1, MM),
                           s1_W, s1_root, s1_b, s2_W, s2_root, s2_b)
    g_sum, g_cnt = _encode(goal_x, goal_edge_index[0], goal_edge_index[1],
                           goal_edge_type, goal_batch.reshape(GG, 1, MM),
                           g1_W, g1_root, g1_b, g2_W, g2_root, g2_b)
    pred = _tc_tail(s_sum, s_cnt.reshape(BB, 1), g_sum, g_cnt.reshape(BB, 1),
                    depth.reshape(BB, 1),
                    reg_W1[:HH], reg_W1[HH:2 * HH], reg_W1[2 * HH:],
                    reg_b1.reshape(1, HH), reg_W2.reshape(1, HH),
                    reg_b2.reshape(1, 1))
    return pred.reshape(BB)


# speedup vs baseline: 3.7241x; 1.0518x over previous
import functools

import jax
import jax.numpy as jnp
from jax import lax
from jax.experimental import pallas as pl
from jax.experimental.pallas import tpu as pltpu
from jax.experimental.pallas import tpu_sc as plsc

NN = 10000
EE = 320000
DD = 128
HH = 64
RR = 32
BB = 64

NC = 2
NS = 16
LL = 16
NW = NC * NS

KK = 80
NB = 5
ET = EE // NS
NGR = ET // (NB * KK)
NRR = NN * RR
ZB = 2000
ZR = 40
NCH = NN // ZR
CPT = -(-NCH // NS)

_mesh = plsc.VectorSubcoreMesh(core_axis_name="c", subcore_axis_name="s",
                               num_cores=NC, num_subcores=NS)
_sc_params = pltpu.CompilerParams(use_tc_tiling_on_sc=False)


def _zero_rows(zrow):
    for r in range(ZR):
        for j in range(HH // LL):
            zrow[r, pl.ds(j * LL, LL)] = jnp.zeros((LL,), jnp.float32)


def _zero_agg(agg, zrow, s):
    def zc(jj, _):
        j = jj * NS + s

        @pl.when(j < NCH)
        def _():
            pltpu.sync_copy(zrow, agg.at[pl.ds(j * ZR, ZR)])
        return 0
    lax.fori_loop(0, CPT, zc, 0)


def _copy_out(agg, aggp_hbm, c, s):
    def oc(jj, _):
        j = jj * NS + s

        @pl.when(j < NCH)
        def _():
            pltpu.sync_copy(agg.at[pl.ds(j * ZR, ZR)],
                            aggp_hbm.at[pl.ds(c * NN + j * ZR, ZR)])
        return 0
    lax.fori_loop(0, CPT, oc, 0)


def _scale_rows(rows_vs, norm_vs, b):
    for j16 in range(KK // LL):
        nv16 = norm_vs[b, pl.ds(j16 * LL, LL)]
        for l in range(LL):
            e = j16 * LL + l
            nv = nv16[l]
            for j in range(HH // LL):
                sl = pl.ds(j * LL, LL)
                rows_vs[b, e, sl] = rows_vs[b, e, sl] * nv


def _sc_l1_body(srcb, dstb, etb, xrel_s, xrel_g,
                combsrc_hbm, normb_hbm, aggp_hbm,
                counts, agg, src_vs, dst_vs, et_vs, comb_vs, comb2_vs,
                ones_v, cnt_vs, norm_vs, rows_vs, zbuf, zrow, *sems):
    lsems = sems[0:NB]
    gsems = sems[NB:2 * NB]
    csems = sems[2 * NB:3 * NB]
    ssems = sems[3 * NB:4 * NB]
    asems = sems[4 * NB:5 * NB]
    c = lax.axis_index("c")
    s = lax.axis_index("s")

    for j in range(KK // LL):
        ones_v[pl.ds(j * LL, LL)] = jnp.ones((LL,), jnp.float32)

    def zfill(i, _):
        zbuf[pl.ds(i * LL, LL)] = jnp.zeros((LL,), jnp.float32)
        return 0
    lax.fori_loop(0, ZB // LL, zfill, 0)

    def zcopy(j, _):
        pltpu.sync_copy(zbuf, counts.at[pl.ds(s * (NRR // NS) + j * ZB, ZB)])
        return 0
    lax.fori_loop(0, (NRR // NS) // ZB, zcopy, 0)
    _zero_rows(zrow)
    _zero_agg(agg, zrow, s)
    plsc.subcore_barrier()

    def p1(g, _):
        base0 = s * ET + g * (NB * KK)
        ldescs = []
        for b in range(NB):
            base = base0 + b * KK
            d1 = pltpu.async_copy(dstb.at[c, pl.ds(base, KK)],
                                  dst_vs.at[b], lsems[b])
            d2 = pltpu.async_copy(etb.at[c, pl.ds(base, KK)],
                                  et_vs.at[b], lsems[b])
            d3 = pltpu.async_copy(srcb.at[c, pl.ds(base, KK)],
                                  src_vs.at[b], lsems[b])
            ldescs.append((d1, d2, d3))
        adescs = []
        for b in range(NB):
            base = base0 + b * KK
            for d in ldescs[b]:
                d.wait()
            for j in range(KK // LL):
                sl = pl.ds(j * LL, LL)
                comb_vs[b, sl] = dst_vs[b, sl] * RR + et_vs[b, sl]
                comb2_vs[b, sl] = src_vs[b, sl] * RR + et_vs[b, sl]
            a1 = pltpu.async_copy(ones_v, counts.at[comb_vs.at[b]],
                                  gsems[b], add=True)
            a2 = pltpu.async_copy(comb2_vs.at[b],
                                  combsrc_hbm.at[c, pl.ds(base, KK)],
                                  csems[b])
            adescs.append((a1, a2))
        for b in range(NB):
            for d in adescs[b]:
                d.wait()
        return 0
    lax.fori_loop(0, NGR, p1, 0)
    plsc.subcore_barrier()

    def p2(g, _):
        base0 = s * ET + g * (NB * KK)
        ldescs = []
        for b in range(NB):
            base = base0 + b * KK
            d1 = pltpu.async_copy(dstb.at[c, pl.ds(base, KK)],
                                  dst_vs.at[b], lsems[b])
            d2 = pltpu.async_copy(etb.at[c, pl.ds(base, KK)],
                                  et_vs.at[b], lsems[b])
            d3 = pltpu.async_copy(srcb.at[c, pl.ds(base, KK)],
                                  src_vs.at[b], lsems[b])
            ldescs.append((d1, d2, d3))
        cdescs = []
        for b in range(NB):
            for d in ldescs[b]:
                d.wait()
            for j in range(KK // LL):
                sl = pl.ds(j * LL, LL)
                comb_vs[b, sl] = dst_vs[b, sl] * RR + et_vs[b, sl]
                comb2_vs[b, sl] = src_vs[b, sl] * RR + et_vs[b, sl]
            cdescs.append(pltpu.async_copy(counts.at[comb_vs.at[b]],
                                           cnt_vs.at[b], gsems[b]))

            @pl.when(c == 0)
            def _(b=b):
                pltpu.async_copy(xrel_s.at[comb2_vs.at[b]],
                                 rows_vs.at[b], csems[b])

            @pl.when(c == 1)
            def _(b=b):
                pltpu.async_copy(xrel_g.at[comb2_vs.at[b]],
                                 rows_vs.at[b], csems[b])
        sdescs = []
        for b in range(NB):
            base = base0 + b * KK
            cdescs[b].wait()
            for j in range(KK // LL):
                sl = pl.ds(j * LL, LL)
                norm_vs[b, sl] = 1.0 / jnp.maximum(cnt_vs[b, sl], 1.0)
            sd = pltpu.async_copy(norm_vs.at[b],
                                  normb_hbm.at[c, pl.ds(base, KK)],
                                  ssems[b])

            @pl.when(c == 0)
            def _(b=b):
                pltpu.make_async_copy(xrel_s.at[comb2_vs.at[b]],
                                      rows_vs.at[b], csems[b]).wait()

            @pl.when(c == 1)
            def _(b=b):
                pltpu.make_async_copy(xrel_g.at[comb2_vs.at[b]],
                                      rows_vs.at[b], csems[b]).wait()
            _scale_rows(rows_vs, norm_vs, b)
            ad = pltpu.async_copy(rows_vs.at[b], agg.at[dst_vs.at[b]],
                                  asems[b], add=True)
            sdescs.append((sd, ad))
        for b in range(NB):
            for d in sdescs[b]:
                d.wait()
        return 0
    lax.fori_loop(0, NGR, p2, 0)
    plsc.subcore_barrier()

    _copy_out(agg, aggp_hbm, c, s)


@functools.partial(
    pl.kernel,
    out_type=[jax.ShapeDtypeStruct((NC, EE), jnp.int32),
              jax.ShapeDtypeStruct((NC, EE), jnp.float32),
              jax.ShapeDtypeStruct((NC * NN, HH), jnp.float32)],
    mesh=_mesh,
    compiler_params=_sc_params,
    scratch_types=[
        pltpu.VMEM_SHARED((NRR,), jnp.float32),
        pltpu.VMEM_SHARED((NN, HH), jnp.float32),
        pltpu.VMEM((NB, KK), jnp.int32),
        pltpu.VMEM((NB, KK), jnp.int32),
        pltpu.VMEM((NB, KK), jnp.int32),
        pltpu.VMEM((NB, KK), jnp.int32),
        pltpu.VMEM((NB, KK), jnp.int32),
        pltpu.VMEM((KK,), jnp.float32),
        pltpu.VMEM((NB, KK), jnp.float32),
        pltpu.VMEM((NB, KK), jnp.float32),
        pltpu.VMEM((NB, KK, HH), jnp.float32),
        pltpu.VMEM((ZB,), jnp.float32),
        pltpu.VMEM((ZR, HH), jnp.float32),
    ] + [pltpu.SemaphoreType.DMA] * (5 * NB),
)
def _sc_l1(srcb, dstb, etb, xrel_s, xrel_g, *rest):
    _sc_l1_body(srcb, dstb, etb, xrel_s, xrel_g, *rest)


def _sc_l2_body(dstb, combsrc_hbm, normb_hbm, xrel_s, xrel_g, aggp_hbm,
                agg, idx_vs, dst_vs, norm_vs, rows_vs, zrow, *sems):
    lsems = sems[0:NB]
    gsems = sems[NB:2 * NB]
    ssems = sems[2 * NB:3 * NB]
    c = lax.axis_index("c")
    s = lax.axis_index("s")

    _zero_rows(zrow)
    _zero_agg(agg, zrow, s)
    plsc.subcore_barrier()

    def group(g, _):
        base0 = s * ET + g * (NB * KK)
        ldescs = []
        for b in range(NB):
            base = base0 + b * KK
            d1 = pltpu.async_copy(combsrc_hbm.at[c, pl.ds(base, KK)],
                                  idx_vs.at[b], lsems[b])
            d2 = pltpu.async_copy(dstb.at[c, pl.ds(base, KK)],
                                  dst_vs.at[b], lsems[b])
            d3 = pltpu.async_copy(normb_hbm.at[c, pl.ds(base, KK)],
                                  norm_vs.at[b], lsems[b])
            ldescs.append((d1, d2, d3))
        for b in range(NB):
            for d in ldescs[b]:
                d.wait()

            @pl.when(c == 0)
            def _(b=b):
                pltpu.async_copy(xrel_s.at[idx_vs.at[b]],
                                 rows_vs.at[b], gsems[b])

            @pl.when(c == 1)
            def _(b=b):
                pltpu.async_copy(xrel_g.at[idx_vs.at[b]],
                                 rows_vs.at[b], gsems[b])
        sdescs = []
        for b in range(NB):
            @pl.when(c == 0)
            def _(b=b):
                pltpu.make_async_copy(xrel_s.at[idx_vs.at[b]],
                                      rows_vs.at[b], gsems[b]).wait()

            @pl.when(c == 1)
            def _(b=b):
                pltpu.make_async_copy(xrel_g.at[idx_vs.at[b]],
                                      rows_vs.at[b], gsems[b]).wait()
            _scale_rows(rows_vs, norm_vs, b)
            sdescs.append(pltpu.async_copy(rows_vs.at[b],
                                           agg.at[dst_vs.at[b]],
                                           ssems[b], add=True))
        for d in sdescs:
            d.wait()
        return 0
    lax.fori_loop(0, NGR, group, 0)
    plsc.subcore_barrier()

    _copy_out(agg, aggp_hbm, c, s)


@functools.partial(
    pl.kernel,
    out_type=jax.ShapeDtypeStruct((NC * NN, HH), jnp.float32),
    mesh=_mesh,
    compiler_params=_sc_params,
    scratch_types=[
        pltpu.VMEM_SHARED((NN, HH), jnp.float32),
        pltpu.VMEM((NB, KK), jnp.int32),
        pltpu.VMEM((NB, KK), jnp.int32),
        pltpu.VMEM((NB, KK), jnp.float32),
        pltpu.VMEM((NB, KK, HH), jnp.float32),
        pltpu.VMEM((ZR, HH), jnp.float32),
    ] + [pltpu.SemaphoreType.DMA] * (3 * NB),
)
def _sc_l2(dstb, combsrc_hbm, normb_hbm, xrel_s, xrel_g, *rest):
    _sc_l2_body(dstb, combsrc_hbm, normb_hbm, xrel_s, xrel_g, *rest)


MM = 400
GG = NN // MM


def _mm1_body(x_ref, wr_ref, rt_ref, xrel_ref, rto_ref):
    x = x_ref[...]
    xrel_ref[...] = jnp.dot(x, wr_ref[...], preferred_element_type=jnp.float32)
    rto_ref[...] = jnp.dot(x, rt_ref[...], preferred_element_type=jnp.float32)


def _tc_mm1(x, wr, root):
    din, dout = wr.shape
    return pl.pallas_call(
        _mm1_body,
        grid=(GG,),
        in_specs=[pl.BlockSpec((MM, din), lambda i: (i, 0)),
                  pl.BlockSpec((din, dout), lambda i: (0, 0)),
                  pl.BlockSpec((din, HH), lambda i: (0, 0))],
        out_specs=[pl.BlockSpec((MM, dout), lambda i: (i, 0)),
                   pl.BlockSpec((MM, HH), lambda i: (i, 0))],
        out_shape=[jax.ShapeDtypeStruct((NN, dout), jnp.float32),
                   jax.ShapeDtypeStruct((NN, HH), jnp.float32)],
    )(x, wr, root)


def _mm2_body(agg_ref, rt1_ref, b1_ref, wr_ref, rt2_ref, xrel_ref, rto_ref):
    h = jnp.maximum(agg_ref[...] + rt1_ref[...] + b1_ref[...], 0.0)
    xrel_ref[...] = jnp.dot(h, wr_ref[...], preferred_element_type=jnp.float32)
    rto_ref[...] = jnp.dot(h, rt2_ref[...], preferred_element_type=jnp.float32)


def _tc_mm2(agg, rt1, b1, wr, root):
    din, dout = wr.shape
    return pl.pallas_call(
        _mm2_body,
        grid=(GG,),
        in_specs=[pl.BlockSpec((MM, HH), lambda i: (i, 0)),
                  pl.BlockSpec((MM, HH), lambda i: (i, 0)),
                  pl.BlockSpec((1, HH), lambda i: (0, 0)),
                  pl.BlockSpec((din, dout), lambda i: (0, 0)),
                  pl.BlockSpec((din, HH), lambda i: (0, 0))],
        out_specs=[pl.BlockSpec((MM, dout), lambda i: (i, 0)),
                   pl.BlockSpec((MM, HH), lambda i: (i, 0))],
        out_shape=[jax.ShapeDtypeStruct((NN, dout), jnp.float32),
                   jax.ShapeDtypeStruct((NN, HH), jnp.float32)],
    )(agg, rt1, b1, wr, root)


def _pool_body(agg_ref, rt2_ref, b2_ref, batch_ref, psum_ref, pcnt_ref):
    i = pl.program_id(0)
    h = jnp.maximum(agg_ref[...] + rt2_ref[...] + b2_ref[...], 0.0)
    bt = batch_ref[0, 0, :]
    oh = (bt[None, :] == lax.broadcasted_iota(jnp.int32, (BB, MM), 0)
          ).astype(jnp.float32)
    ps = jnp.dot(oh, h, preferred_element_type=jnp.float32)
    pc = jnp.sum(oh, axis=1)[None, :]

    @pl.when(i == 0)
    def _():
        psum_ref[...] = jnp.zeros_like(psum_ref)
        pcnt_ref[...] = jnp.zeros_like(pcnt_ref)
    psum_ref[...] += ps
    pcnt_ref[...] += pc


def _tc_pool(agg, rt2, b2, batch3d):
    return pl.pallas_call(
        _pool_body,
        grid=(GG,),
        in_specs=[pl.BlockSpec((MM, HH), lambda i: (i, 0)),
                  pl.BlockSpec((MM, HH), lambda i: (i, 0)),
                  pl.BlockSpec((1, HH), lambda i: (0, 0)),
                  pl.BlockSpec((1, 1, MM), lambda i: (i, 0, 0))],
        out_specs=[pl.BlockSpec((BB, HH), lambda i: (0, 0)),
                   pl.BlockSpec((1, BB), lambda i: (0, 0))],
        out_shape=[jax.ShapeDtypeStruct((BB, HH), jnp.float32),
                   jax.ShapeDtypeStruct((1, BB), jnp.float32)],
    )(agg, rt2, b2, batch3d)


def _tail_body(sps_ref, spc_ref, gps_ref, gpc_ref, d_ref, w1a_ref, w1b_ref,
               w1c_ref, b1_ref, w2r_ref, b2_ref, out_ref):
    se = sps_ref[...] / jnp.maximum(spc_ref[...], 1.0)
    ge = gps_ref[...] / jnp.maximum(gpc_ref[...], 1.0)
    d = d_ref[...]
    dm = jnp.mean(d)
    sd = jnp.sqrt(jnp.mean((d - dm) ** 2))
    dn = (d - dm) / (sd + 1e-6)
    z = (jnp.dot(se, w1a_ref[...], preferred_element_type=jnp.float32)
         + jnp.dot(ge, w1b_ref[...], preferred_element_type=jnp.float32)
         + dn * w1c_ref[...] + b1_ref[...])
    hh = jnp.maximum(z, 0.0)
    out_ref[...] = jnp.sum(hh * w2r_ref[...], axis=1, keepdims=True) \
        + b2_ref[...]


def _tc_tail(sps, spc, gps, gpc, d, w1a, w1b, w1c, b1, w2r, b2):
    return pl.pallas_call(
        _tail_body,
        out_shape=jax.ShapeDtypeStruct((BB, 1), jnp.float32),
    )(sps, spc, gps, gpc, d, w1a, w1b, w1c, b1, w2r, b2)


def _wr(W):
    return W.transpose(1, 0, 2).reshape(W.shape[1], RR * HH)


def kernel(state_x, state_edge_index, state_edge_type, state_batch,
           goal_x, goal_edge_index, goal_edge_type, goal_batch, depth,
           s1_W, s1_root, s1_b, s2_W, s2_root, s2_b,
           g1_W, g1_root, g1_b, g2_W, g2_root, g2_b,
           reg_W1, reg_b1, reg_W2, reg_b2):
    srcb = jnp.stack([state_edge_index[0], goal_edge_index[0]])
    dstb = jnp.stack([state_edge_index[1], goal_edge_index[1]])
    etb = jnp.stack([state_edge_type, goal_edge_type])

    xrel1_s, rt1_s = _tc_mm1(state_x, _wr(s1_W), s1_root)
    xrel1_g, rt1_g = _tc_mm1(goal_x, _wr(g1_W), g1_root)
    combsrc, normb, agg1 = _sc_l1(srcb, dstb, etb,
                                  xrel1_s.reshape(NRR, HH),
                                  xrel1_g.reshape(NRR, HH))
    xrel2_s, rt2_s = _tc_mm2(agg1[:NN], rt1_s, s1_b.reshape(1, HH),
                             _wr(s2_W), s2_root)
    xrel2_g, rt2_g = _tc_mm2(agg1[NN:], rt1_g, g1_b.reshape(1, HH),
                             _wr(g2_W), g2_root)
    agg2 = _sc_l2(dstb, combsrc, normb,
                  xrel2_s.reshape(NRR, HH), xrel2_g.reshape(NRR, HH))
    s_sum, s_cnt = _tc_pool(agg2[:NN], rt2_s, s2_b.reshape(1, HH),
                            state_batch.reshape(GG, 1, MM))
    g_sum, g_cnt = _tc_pool(agg2[NN:], rt2_g, g2_b.reshape(1, HH),
                            goal_batch.reshape(GG, 1, MM))
    pred = _tc_tail(s_sum, s_cnt.reshape(BB, 1), g_sum, g_cnt.reshape(BB, 1),
                    depth.reshape(BB, 1),
                    reg_W1[:HH], reg_W1[HH:2 * HH], reg_W1[2 * HH:],
                    reg_b1.reshape(1, HH), reg_W2.reshape(1, HH),
                    reg_b2.reshape(1, 1))
    return pred.reshape(BB)
